# Initial kernel scaffold; baseline (speedup 1.0000x reference)
import jax
import jax.numpy as jnp
from jax.experimental import pallas as pl


def kernel(x, edge_index, W1, b1, Wg, att_src, att_dst, bg, Wfc, bfc):
    raise NotImplementedError("write your pallas kernel here")


# speedup vs baseline: 3.7649x; 3.7649x over previous
import functools

import jax
import jax.numpy as jnp
from jax.experimental import pallas as pl

_N = 10000
_HEADS = 4
_D_OUT = 64
_BLK = 1000


def _stage1_body(x_ref, w1_ref, deg_ref, out_ref):
    h = jax.lax.dot_general(
        x_ref[...], w1_ref[...], (((1,), (0,)), ((), ())),
        preferred_element_type=jnp.float32)
    deg = deg_ref[...]
    dinv = jnp.where(deg > 0, jax.lax.rsqrt(deg), 0.0)
    out_ref[...] = h * dinv


def _stage2_body(hagg_ref, deg_ref, b1_ref, wg_ref, att_ref, g_ref, a_ref):
    deg = deg_ref[...]
    dinv = jnp.where(deg > 0, jax.lax.rsqrt(deg), 0.0)
    h = hagg_ref[...] * dinv + b1_ref[...]
    h = jnp.where(h >= 0, h, 0.01 * h)
    g = jax.lax.dot_general(
        h, wg_ref[...], (((1,), (0,)), ((), ())),
        preferred_element_type=jnp.float32)
    g_ref[...] = g
    a_ref[...] = jax.lax.dot_general(
        g, att_ref[...], (((1,), (0,)), ((), ())),
        preferred_element_type=jnp.float32)


def _stage3_body(agg_ref, den_ref, bg_ref, wfc_ref, bfc_ref, out_ref):
    den = jnp.maximum(den_ref[...], 1e-16)
    acc = jnp.zeros((agg_ref.shape[0], _D_OUT), jnp.float32)
    for h in range(_HEADS):
        scale = (1.0 / _HEADS) / den[:, h:h + 1]
        acc = acc + agg_ref[:, h * _D_OUT:(h + 1) * _D_OUT] * scale
    t = acc + bg_ref[...]
    t = jnp.where(t >= 0, t, 0.01 * t)
    out_ref[...] = jax.lax.dot_general(
        t, wfc_ref[...], (((1,), (0,)), ((), ())),
        preferred_element_type=jnp.float32) + bfc_ref[...]


def _rows_spec(width):
    return pl.BlockSpec((_BLK, width), lambda i: (i, 0))


def _full_spec(r, c):
    return pl.BlockSpec((r, c), lambda i: (0, 0))


@jax.jit
def kernel(x, edge_index, W1, b1, Wg, att_src, att_dst, bg, Wfc, bfc):
    n = x.shape[0]
    d_in = x.shape[1]
    d_hid = W1.shape[1]
    d_gat = Wg.shape[1]

    loop = jnp.arange(n, dtype=edge_index.dtype)
    src = jnp.concatenate([edge_index[0], loop])
    dst = jnp.concatenate([edge_index[1], loop])

    deg = jax.ops.segment_sum(jnp.ones_like(src, jnp.float32), dst,
                              num_segments=n).reshape(n, 1)

    grid = (n // _BLK,)

    h_scaled = pl.pallas_call(
        _stage1_body,
        grid=grid,
        in_specs=[_rows_spec(d_in), _full_spec(d_in, d_hid), _rows_spec(1)],
        out_specs=_rows_spec(d_hid),
        out_shape=jax.ShapeDtypeStruct((n, d_hid), jnp.float32),
    )(x, W1, deg)

    hagg = jax.ops.segment_sum(h_scaled[src], dst, num_segments=n)

    eye_h = jnp.eye(_HEADS, dtype=jnp.float32)
    a_src_mat = (att_src[:, :, None] * eye_h[:, None, :]).reshape(d_gat, _HEADS)
    a_dst_mat = (att_dst[:, :, None] * eye_h[:, None, :]).reshape(d_gat, _HEADS)
    att_mat = jnp.concatenate([a_src_mat, a_dst_mat], axis=1)

    g, a = pl.pallas_call(
        _stage2_body,
        grid=grid,
        in_specs=[_rows_spec(d_hid), _rows_spec(1), _full_spec(1, d_hid),
                  _full_spec(d_hid, d_gat), _full_spec(d_gat, 2 * _HEADS)],
        out_specs=[_rows_spec(d_gat), _rows_spec(2 * _HEADS)],
        out_shape=[jax.ShapeDtypeStruct((n, d_gat), jnp.float32),
                   jax.ShapeDtypeStruct((n, 2 * _HEADS), jnp.float32)],
    )(hagg, deg, b1.reshape(1, d_hid), Wg, att_mat)
    a_src_n = a[:, :_HEADS]
    a_dst_n = a[:, _HEADS:]

    alpha = a_src_n[src] + a_dst_n[dst]
    alpha = jnp.where(alpha >= 0, alpha, 0.2 * alpha)
    amax = jax.ops.segment_max(alpha, dst, num_segments=n)
    alpha = jnp.exp(alpha - amax[dst])
    denom = jax.ops.segment_sum(alpha, dst, num_segments=n)
    gsrc = g[src].reshape(-1, _HEADS, _D_OUT)
    msg = (gsrc * alpha[:, :, None]).reshape(-1, d_gat)
    agg = jax.ops.segment_sum(msg, dst, num_segments=n)

    out = pl.pallas_call(
        _stage3_body,
        grid=grid,
        in_specs=[_rows_spec(d_gat), _rows_spec(_HEADS), _full_spec(1, _D_OUT),
                  _full_spec(_D_OUT, 1), _full_spec(1, 1)],
        out_specs=_rows_spec(1),
        out_shape=jax.ShapeDtypeStruct((n, 1), jnp.float32),
    )(agg, denom, bg.reshape(1, _D_OUT), Wfc, bfc.reshape(1, 1))
    return out.reshape(n)


# speedup vs baseline: 4.9097x; 1.3041x over previous
import functools

import jax
import jax.numpy as jnp
from jax.experimental import pallas as pl

_N = 10000
_HEADS = 4
_D_OUT = 64
_BLK = 1000


def _stage1_body(x_ref, deg_ref, out_ref):
    deg = deg_ref[...]
    dinv = jnp.where(deg > 0, jax.lax.rsqrt(deg), 0.0)
    out_ref[...] = x_ref[...] * dinv


def _stage2_body(xagg_ref, deg_ref, w1_ref, b1_ref, wg_ref, att_ref,
                 g_ref, a_ref):
    deg = deg_ref[...]
    dinv = jnp.where(deg > 0, jax.lax.rsqrt(deg), 0.0)
    h = jax.lax.dot_general(
        xagg_ref[...], w1_ref[...], (((1,), (0,)), ((), ())),
        preferred_element_type=jnp.float32) * dinv + b1_ref[...]
    h = jnp.where(h >= 0, h, 0.01 * h)
    g = jax.lax.dot_general(
        h, wg_ref[...], (((1,), (0,)), ((), ())),
        preferred_element_type=jnp.float32)
    g_ref[...] = g
    a_ref[...] = jax.lax.dot_general(
        g, att_ref[...], (((1,), (0,)), ((), ())),
        preferred_element_type=jnp.float32)


def _stage3_body(agg_ref, den_ref, bg_ref, wfc_ref, bfc_ref, out_ref):
    den = jnp.maximum(den_ref[...], 1e-16)
    acc = jnp.zeros((agg_ref.shape[0], _D_OUT), jnp.float32)
    for h in range(_HEADS):
        scale = (1.0 / _HEADS) / den[:, h:h + 1]
        acc = acc + agg_ref[:, h * _D_OUT:(h + 1) * _D_OUT] * scale
    t = acc + bg_ref[...]
    t = jnp.where(t >= 0, t, 0.01 * t)
    out_ref[...] = jax.lax.dot_general(
        t, wfc_ref[...], (((1,), (0,)), ((), ())),
        preferred_element_type=jnp.float32) + bfc_ref[...]


def _rows_spec(width):
    return pl.BlockSpec((_BLK, width), lambda i: (i, 0))


def _full_spec(r, c):
    return pl.BlockSpec((r, c), lambda i: (0, 0))


@jax.jit
def kernel(x, edge_index, W1, b1, Wg, att_src, att_dst, bg, Wfc, bfc):
    n = x.shape[0]
    d_in = x.shape[1]
    d_hid = W1.shape[1]
    d_gat = Wg.shape[1]

    loop = jnp.arange(n, dtype=edge_index.dtype)
    src = jnp.concatenate([edge_index[0], loop])
    dst = jnp.concatenate([edge_index[1], loop])

    deg = jax.ops.segment_sum(jnp.ones_like(src, jnp.float32), dst,
                              num_segments=n).reshape(n, 1)

    grid = (n // _BLK,)

    x_scaled = pl.pallas_call(
        _stage1_body,
        grid=grid,
        in_specs=[_rows_spec(d_in), _rows_spec(1)],
        out_specs=_rows_spec(d_in),
        out_shape=jax.ShapeDtypeStruct((n, d_in), jnp.float32),
    )(x, deg)

    xagg = jax.ops.segment_sum(x_scaled[src], dst, num_segments=n)

    eye_h = jnp.eye(_HEADS, dtype=jnp.float32)
    a_src_mat = (att_src[:, :, None] * eye_h[:, None, :]).reshape(d_gat, _HEADS)
    a_dst_mat = (att_dst[:, :, None] * eye_h[:, None, :]).reshape(d_gat, _HEADS)
    att_mat = jnp.concatenate([a_src_mat, a_dst_mat], axis=1)

    g, a = pl.pallas_call(
        _stage2_body,
        grid=grid,
        in_specs=[_rows_spec(d_in), _rows_spec(1), _full_spec(d_in, d_hid),
                  _full_spec(1, d_hid), _full_spec(d_hid, d_gat),
                  _full_spec(d_gat, 2 * _HEADS)],
        out_specs=[_rows_spec(d_gat), _rows_spec(2 * _HEADS)],
        out_shape=[jax.ShapeDtypeStruct((n, d_gat), jnp.float32),
                   jax.ShapeDtypeStruct((n, 2 * _HEADS), jnp.float32)],
    )(xagg, deg, W1, b1.reshape(1, d_hid), Wg, att_mat)
    a_src_n = a[:, :_HEADS]
    a_dst_n = a[:, _HEADS:]

    alpha = a_src_n[src] + a_dst_n[dst]
    alpha = jnp.where(alpha >= 0, alpha, 0.2 * alpha)
    amax = jax.ops.segment_max(alpha, dst, num_segments=n)
    alpha = jnp.exp(alpha - amax[dst])
    denom = jax.ops.segment_sum(alpha, dst, num_segments=n)
    gsrc = g[src].reshape(-1, _HEADS, _D_OUT)
    msg = (gsrc * alpha[:, :, None]).reshape(-1, d_gat)
    agg = jax.ops.segment_sum(msg, dst, num_segments=n)

    out = pl.pallas_call(
        _stage3_body,
        grid=grid,
        in_specs=[_rows_spec(d_gat), _rows_spec(_HEADS), _full_spec(1, _D_OUT),
                  _full_spec(_D_OUT, 1), _full_spec(1, 1)],
        out_specs=_rows_spec(1),
        out_shape=jax.ShapeDtypeStruct((n, 1), jnp.float32),
    )(agg, denom, bg.reshape(1, _D_OUT), Wfc, bfc.reshape(1, 1))
    return out.reshape(n)
